# baseline (speedup 1.0000x reference)
import jax
import jax.numpy as jnp
from jax.experimental import pallas as pl
from jax.experimental.pallas import tpu as pltpu


def _add_kernel(x_ref, p_ref, o_ref):
    o_ref[...] = x_ref[...] + p_ref[...][None, :, :]


def kernel(inputs, pos_table):
    batch, seq_len, d_model = inputs.shape
    blk_s = 1024
    grid = (seq_len // blk_s, batch)
    return pl.pallas_call(
        _add_kernel,
        grid=grid,
        in_specs=[
            pl.BlockSpec((1, blk_s, d_model), lambda i, j: (j, i, 0)),
            pl.BlockSpec((blk_s, d_model), lambda i, j: (i, 0)),
        ],
        out_specs=pl.BlockSpec((1, blk_s, d_model), lambda i, j: (j, i, 0)),
        out_shape=jax.ShapeDtypeStruct(inputs.shape, inputs.dtype),
        compiler_params=pltpu.CompilerParams(vmem_limit_bytes=60 * 1024 * 1024),
    )(inputs, pos_table)
